# baseline (device time: 25597 ns/iter reference)
import jax
import jax.numpy as jnp
from jax import lax
from jax.experimental import pallas as pl
from jax.experimental.pallas import tpu as pltpu

N_DEV = 4
N_HALF = 2
QUANT_SCALE = 160.0 / 127.0


def kernel(A, B):
    m, k = A.shape
    _, n = B.shape
    m_chunk = m // N_DEV
    n_half = n // N_HALF

    def body(a_ref, b_ref, out_ref, send_buf, recv_buf, send_sems, recv_sems):
        my_pos = lax.axis_index("i")

        barrier_sem = pltpu.get_barrier_semaphore()
        for j in range(1, N_DEV):
            pl.semaphore_signal(
                barrier_sem, inc=1,
                device_id=(lax.rem(my_pos + j, N_DEV),),
                device_id_type=pl.DeviceIdType.MESH,
            )
        pl.semaphore_wait(barrier_sem, N_DEV - 1)

        def partial_half(c, h):
            a_bf = a_ref[pl.ds(c * m_chunk, m_chunk), :].astype(jnp.bfloat16)
            b_bf = b_ref[:, h * n_half:(h + 1) * n_half].astype(jnp.bfloat16)
            return lax.dot_general(
                a_bf, b_bf,
                (((1,), (0,)), ((), ())),
                preferred_element_type=jnp.float32,
            )

        def quantize(p):
            q = jnp.round(p * (1.0 / QUANT_SCALE))
            return jnp.clip(q, -127.0, 127.0).astype(jnp.int8)

        rdmas = []
        for h in range(N_HALF):
            for j in (2, 1, 3):
                dest = lax.rem(my_pos + j, N_DEV)
                send_buf[j - 1, h] = quantize(partial_half(dest, h))
                rdma = pltpu.make_async_remote_copy(
                    src_ref=send_buf.at[j - 1, h],
                    dst_ref=recv_buf.at[N_DEV - 1 - j, h],
                    send_sem=send_sems.at[j - 1, h],
                    recv_sem=recv_sems.at[N_DEV - 1 - j, h],
                    device_id=(dest,),
                    device_id_type=pl.DeviceIdType.MESH,
                )
                rdma.start()
                rdmas.append(rdma)

        for h in range(N_HALF):
            acc = partial_half(my_pos, h)
            for i in range(3):
                rdmas[3 * h + i].wait_recv()
            for slot in (1, 2, 0):
                acc = acc + recv_buf[slot, h].astype(jnp.float32) * QUANT_SCALE
            out_ref[:, pl.ds(h * n_half, n_half)] = acc
        for rdma in rdmas:
            rdma.wait_send()

    return pl.pallas_call(
        body,
        out_shape=jax.ShapeDtypeStruct((m_chunk, n), jnp.float32),
        in_specs=[
            pl.BlockSpec(memory_space=pltpu.VMEM),
            pl.BlockSpec(memory_space=pltpu.VMEM),
        ],
        out_specs=pl.BlockSpec(memory_space=pltpu.VMEM),
        scratch_shapes=[
            pltpu.VMEM((N_DEV - 1, N_HALF, m_chunk, n_half), jnp.int8),
            pltpu.VMEM((N_DEV - 1, N_HALF, m_chunk, n_half), jnp.int8),
            pltpu.SemaphoreType.DMA((N_DEV - 1, N_HALF)),
            pltpu.SemaphoreType.DMA((N_DEV - 1, N_HALF)),
        ],
        compiler_params=pltpu.CompilerParams(collective_id=0),
    )(A, B)


# device time: 22790 ns/iter; 1.1232x vs baseline; 1.1232x over previous
import jax
import jax.numpy as jnp
from jax import lax
from jax.experimental import pallas as pl
from jax.experimental.pallas import tpu as pltpu

N_DEV = 4
N_SPLIT = 4
QUANT_SCALE = 160.0 / 127.0
RDIR = (1, 1, -1, -1)


def kernel(A, B):
    m, k = A.shape
    _, n = B.shape
    m_chunk = m // N_DEV
    n_piece = n // N_SPLIT

    def body(a_ref, b_ref, out_ref,
             dir_out, diag_out, comb_out, dir_in, relay_in, comb_in,
             s_dir, s_diag, s_comb, r_dir, r_relay, r_comb):
        my_pos = lax.axis_index("i")

        def dev(off):
            return (lax.rem(my_pos + (off % N_DEV), N_DEV),)

        barrier_sem = pltpu.get_barrier_semaphore()
        for off in (1, 3):
            pl.semaphore_signal(
                barrier_sem, inc=1,
                device_id=dev(off), device_id_type=pl.DeviceIdType.MESH,
            )
        pl.semaphore_wait(barrier_sem, 2)

        def partial_piece(c, h):
            a_bf = a_ref[pl.ds(c * m_chunk, m_chunk), :].astype(jnp.bfloat16)
            b_bf = b_ref[:, h * n_piece:(h + 1) * n_piece].astype(jnp.bfloat16)
            return lax.dot_general(
                a_bf, b_bf,
                (((1,), (0,)), ((), ())),
                preferred_element_type=jnp.float32,
            )

        def pp(off, h):
            return partial_piece(lax.rem(my_pos + (off % N_DEV), N_DEV), h)

        def quantize(p):
            q = jnp.round(p * (1.0 / QUANT_SCALE))
            return jnp.clip(q, -127.0, 127.0).astype(jnp.int8)

        def dequant(q):
            return q.astype(jnp.float32) * QUANT_SCALE

        def mk(src, dst, ssem, rsem, off):
            return pltpu.make_async_remote_copy(
                src_ref=src, dst_ref=dst, send_sem=ssem, recv_sem=rsem,
                device_id=dev(off), device_id_type=pl.DeviceIdType.MESH,
            )

        diag_rd = []
        for h in range(N_SPLIT):
            diag_out[h] = quantize(pp(2, h))
            rd = mk(diag_out.at[h], relay_in.at[h],
                    s_diag.at[h], r_relay.at[h], 2 + RDIR[h])
            rd.start()
            diag_rd.append(rd)

        dir_rd, comb_rd = [], []
        for h in range(N_SPLIT):
            r = RDIR[h]
            dir_out[h] = quantize(pp(r, h))
            rd = mk(dir_out.at[h], dir_in.at[h], s_dir.at[h], r_dir.at[h], r)
            rd.start()
            dir_rd.append(rd)

            diag_rd[h].wait_recv()
            comb_out[h] = quantize(dequant(relay_in[h]) + pp(-r, h))
            rc = mk(comb_out.at[h], comb_in.at[h],
                    s_comb.at[h], r_comb.at[h], -r)
            rc.start()
            comb_rd.append(rc)

        for h in range(N_SPLIT):
            own = partial_piece(my_pos, h)
            dir_rd[h].wait_recv()
            comb_rd[h].wait_recv()
            out_ref[:, pl.ds(h * n_piece, n_piece)] = (
                own + dequant(dir_in[h]) + dequant(comb_in[h])
            )

        for rd in diag_rd + dir_rd + comb_rd:
            rd.wait_send()

    piece_buf = pltpu.VMEM((N_SPLIT, m_chunk, n_piece), jnp.int8)
    sems = pltpu.SemaphoreType.DMA((N_SPLIT,))
    return pl.pallas_call(
        body,
        out_shape=jax.ShapeDtypeStruct((m_chunk, n), jnp.float32),
        in_specs=[
            pl.BlockSpec(memory_space=pltpu.VMEM),
            pl.BlockSpec(memory_space=pltpu.VMEM),
        ],
        out_specs=pl.BlockSpec(memory_space=pltpu.VMEM),
        scratch_shapes=[piece_buf] * 6 + [sems] * 6,
        compiler_params=pltpu.CompilerParams(collective_id=0),
    )(A, B)
